# Initial kernel scaffold; baseline (speedup 1.0000x reference)
import jax
import jax.numpy as jnp
from jax.experimental import pallas as pl


def kernel(forest0, forest1, forest2, feature_matrix, W1, W2):
    raise NotImplementedError("write your pallas kernel here")


# speedup vs baseline: 2.9710x; 2.9710x over previous
import functools

import jax
import jax.numpy as jnp
from jax import lax
from jax.experimental import pallas as pl
from jax.experimental.pallas import tpu as pltpu
from jax.experimental.pallas import tpu_sc as plsc

_NC = 2
_NS = 16
_NW = _NC * _NS


def _sc_gather_all(forest0, forest1f, forest2f, fm):
    B = forest0.shape[0]
    N1 = forest1f.shape[0]
    N2 = forest2f.shape[0]
    F = fm.shape[1]
    S2 = N2 // N1
    n0 = B // _NW
    n1 = N1 // _NW
    CH = 8
    ROWS = CH * S2
    nchunk1 = n1 // ROWS
    nseg = (N2 // S2) // _NW
    nchunk2 = nseg // CH

    mesh = plsc.VectorSubcoreMesh(core_axis_name="c", subcore_axis_name="s")

    @functools.partial(
        pl.kernel,
        mesh=mesh,
        out_type=[
            jax.ShapeDtypeStruct((B, F), jnp.float32),
            jax.ShapeDtypeStruct((N1, F), jnp.float32),
            jax.ShapeDtypeStruct((N1, F), jnp.float32),
        ],
        scratch_types=[
            pltpu.VMEM((n0,), jnp.int32),
            pltpu.VMEM((ROWS,), jnp.int32),
            pltpu.VMEM((ROWS, F), jnp.float32),
            pltpu.VMEM((CH, F), jnp.float32),
            pltpu.SemaphoreType.DMA,
        ],
    )
    def sc_kernel(f0_hbm, f1_hbm, f2_hbm, fm_hbm, out0, out1, out2,
                  idx0_v, idx_v, rows_v, acc_v, sem):
        wid = lax.axis_index("s") * _NC + lax.axis_index("c")

        base0 = wid * n0
        pltpu.sync_copy(f0_hbm.at[pl.ds(base0, n0)], idx0_v)
        pltpu.async_copy(fm_hbm.at[idx0_v], rows_v.at[pl.ds(0, n0)], sem).wait()
        pltpu.sync_copy(rows_v.at[pl.ds(0, n0)], out0.at[pl.ds(base0, n0)])

        def f1_chunk(c, carry):
            base = wid * n1 + c * ROWS
            pltpu.sync_copy(f1_hbm.at[pl.ds(base, ROWS)], idx_v)
            pltpu.async_copy(fm_hbm.at[idx_v], rows_v, sem).wait()
            pltpu.sync_copy(rows_v, out1.at[pl.ds(base, ROWS)])
            return carry
        lax.fori_loop(0, nchunk1, f1_chunk, 0)

        def x2_chunk(c, carry):
            rbase = (wid * nseg + c * CH) * S2
            sbase = wid * nseg + c * CH
            pltpu.sync_copy(f2_hbm.at[pl.ds(rbase, ROWS)], idx_v)
            pltpu.async_copy(fm_hbm.at[idx_v], rows_v, sem).wait()

            def seg(s, inner):
                r0 = s * S2
                for kk in range(F // 16):
                    col = kk * 16
                    a = rows_v[r0, pl.ds(col, 16)]
                    for j in range(1, S2):
                        a = a + rows_v[r0 + j, pl.ds(col, 16)]
                    acc_v[s, pl.ds(col, 16)] = a
                return inner
            lax.fori_loop(0, CH, seg, 0)
            pltpu.sync_copy(acc_v, out2.at[pl.ds(sbase, CH)])
            return carry
        lax.fori_loop(0, nchunk2, x2_chunk, 0)

    return sc_kernel(forest0, forest1f, forest2f, fm)


def _tc_layer1(feat1, x2s, feat0, w1a, w1b16):
    N1, F = feat1.shape
    B = feat0.shape[0]
    R = 2048
    G = R // 16
    grid = N1 // R

    def body(f1_ref, x2_ref, f0_ref, wa_ref, wb_ref, h0_ref, h1s_ref):
        f1 = f1_ref[...]
        wa = wa_ref[...]
        wb = wb_ref[...]
        h1 = jnp.dot(f1, wa, preferred_element_type=jnp.float32)
        h1 = h1 + jnp.dot(x2_ref[...], wb, preferred_element_type=jnp.float32)
        h1 = jnp.maximum(h1, 0.0)
        h1s_ref[...] = h1.reshape(G, 16, F).sum(axis=1)
        xs = f1.reshape(G, 16, F).sum(axis=1)
        h0 = jnp.dot(f0_ref[...], wa, preferred_element_type=jnp.float32)
        h0 = h0 + jnp.dot(xs, wb, preferred_element_type=jnp.float32)
        h0_ref[...] = jnp.maximum(h0, 0.0)

    return pl.pallas_call(
        body,
        grid=(grid,),
        in_specs=[
            pl.BlockSpec((R, F), lambda i: (i, 0)),
            pl.BlockSpec((R, F), lambda i: (i, 0)),
            pl.BlockSpec((G, F), lambda i: (i, 0)),
            pl.BlockSpec((F, F), lambda i: (0, 0)),
            pl.BlockSpec((F, F), lambda i: (0, 0)),
        ],
        out_specs=[
            pl.BlockSpec((G, F), lambda i: (i, 0)),
            pl.BlockSpec((G, F), lambda i: (i, 0)),
        ],
        out_shape=[
            jax.ShapeDtypeStruct((B, F), jnp.float32),
            jax.ShapeDtypeStruct((B, F), jnp.float32),
        ],
    )(feat1, x2s, feat0, w1a, w1b16)


def _tc_layer2(h0, h1s, w2a, w2b16):
    B, H = h0.shape

    def body(h0_ref, h1_ref, wa_ref, wb_ref, out_ref):
        o = jnp.dot(h0_ref[...], wa_ref[...], preferred_element_type=jnp.float32)
        o = o + jnp.dot(h1_ref[...], wb_ref[...], preferred_element_type=jnp.float32)
        out_ref[...] = jnp.maximum(o, 0.0)

    return pl.pallas_call(
        body,
        out_shape=jax.ShapeDtypeStruct((B, H), jnp.float32),
    )(h0, h1s, w2a, w2b16)


def kernel(forest0, forest1, forest2, feature_matrix, W1, W2):
    F = feature_matrix.shape[1]
    f0 = forest0.astype(jnp.int32)
    f1 = forest1.reshape(-1).astype(jnp.int32)
    f2 = forest2.reshape(-1).astype(jnp.int32)

    feat0, feat1, x2s = _sc_gather_all(f0, f1, f2, feature_matrix)

    W1t = W1.T
    w1a = W1t[:F]
    w1b16 = W1t[F:] * (1.0 / 16.0)
    W2t = W2.T
    w2a = W2t[:F]
    w2b16 = W2t[F:] * (1.0 / 16.0)

    h0, h1s = _tc_layer1(feat1, x2s, feat0, w1a, w1b16)
    return _tc_layer2(h0, h1s, w2a, w2b16)


# speedup vs baseline: 4.2152x; 1.4188x over previous
import functools

import jax
import jax.numpy as jnp
from jax import lax
from jax.experimental import pallas as pl
from jax.experimental.pallas import tpu as pltpu
from jax.experimental.pallas import tpu_sc as plsc

_NC = 2
_NS = 16
_NW = _NC * _NS


def _sc_gather_all(forest0, forest1f, forest2f, fm):
    B = forest0.shape[0]
    N1 = forest1f.shape[0]
    N2 = forest2f.shape[0]
    F = fm.shape[1]
    S2 = N2 // N1
    n0 = B // _NW
    n1 = N1 // _NW
    CH = 8
    ROWS = CH * S2
    nchunk1 = n1 // ROWS
    nseg = (N2 // S2) // _NW
    nchunk2 = nseg // CH

    mesh = plsc.VectorSubcoreMesh(core_axis_name="c", subcore_axis_name="s")

    @functools.partial(
        pl.kernel,
        mesh=mesh,
        out_type=[
            jax.ShapeDtypeStruct((B, F), jnp.float32),
            jax.ShapeDtypeStruct((N1, F), jnp.float32),
            jax.ShapeDtypeStruct((N1, F), jnp.float32),
        ],
        scratch_types=[
            pltpu.VMEM((n0,), jnp.int32),
            pltpu.VMEM((ROWS,), jnp.int32),
            pltpu.VMEM((ROWS,), jnp.int32),
            pltpu.VMEM((ROWS, F), jnp.float32),
            pltpu.VMEM((ROWS, F), jnp.float32),
            pltpu.VMEM((CH, F), jnp.float32),
            pltpu.SemaphoreType.DMA,
            pltpu.SemaphoreType.DMA,
        ],
    )
    def sc_kernel(f0_hbm, f1_hbm, f2_hbm, fm_hbm, out0, out1, out2,
                  idx0_v, idxa_v, idxb_v, bufa_v, bufb_v, acc_v, sema, semb):
        wid = lax.axis_index("s") * _NC + lax.axis_index("c")

        base0 = wid * n0
        pltpu.sync_copy(f0_hbm.at[pl.ds(base0, n0)], idx0_v)
        pltpu.async_copy(fm_hbm.at[idx0_v], bufa_v.at[pl.ds(0, n0)], sema).wait()
        pltpu.sync_copy(bufa_v.at[pl.ds(0, n0)], out0.at[pl.ds(base0, n0)])

        def f1_issue(c, idx_v, buf_v, sem):
            @pl.when(c < nchunk1)
            def _():
                base = wid * n1 + c * ROWS
                pltpu.sync_copy(f1_hbm.at[pl.ds(base, ROWS)], idx_v)
                pltpu.async_copy(fm_hbm.at[idx_v], buf_v, sem)

        def f1_drain(c, idx_v, buf_v, sem):
            pltpu.make_async_copy(fm_hbm.at[idx_v], buf_v, sem).wait()
            pltpu.sync_copy(buf_v, out1.at[pl.ds(wid * n1 + c * ROWS, ROWS)])

        f1_issue(0, idxa_v, bufa_v, sema)

        def f1_pair(p, carry):
            c0 = 2 * p
            f1_issue(c0 + 1, idxb_v, bufb_v, semb)
            f1_drain(c0, idxa_v, bufa_v, sema)
            f1_issue(c0 + 2, idxa_v, bufa_v, sema)
            f1_drain(c0 + 1, idxb_v, bufb_v, semb)
            return carry
        lax.fori_loop(0, nchunk1 // 2, f1_pair, 0)

        def x2_issue(c, idx_v, buf_v, sem):
            @pl.when(c < nchunk2)
            def _():
                rbase = (wid * nseg + c * CH) * S2
                pltpu.sync_copy(f2_hbm.at[pl.ds(rbase, ROWS)], idx_v)
                pltpu.async_copy(fm_hbm.at[idx_v], buf_v, sem)

        def x2_drain(c, idx_v, buf_v, sem):
            pltpu.make_async_copy(fm_hbm.at[idx_v], buf_v, sem).wait()

            def seg(s, inner):
                r0 = s * S2
                for kk in range(F // 16):
                    col = kk * 16
                    a = buf_v[r0, pl.ds(col, 16)]
                    for j in range(1, S2):
                        a = a + buf_v[r0 + j, pl.ds(col, 16)]
                    acc_v[s, pl.ds(col, 16)] = a
                return inner
            lax.fori_loop(0, CH, seg, 0)
            pltpu.sync_copy(acc_v, out2.at[pl.ds(wid * nseg + c * CH, CH)])

        x2_issue(0, idxa_v, bufa_v, sema)

        def x2_pair(p, carry):
            c0 = 2 * p
            x2_issue(c0 + 1, idxb_v, bufb_v, semb)
            x2_drain(c0, idxa_v, bufa_v, sema)
            x2_issue(c0 + 2, idxa_v, bufa_v, sema)
            x2_drain(c0 + 1, idxb_v, bufb_v, semb)
            return carry
        lax.fori_loop(0, nchunk2 // 2, x2_pair, 0)

    return sc_kernel(forest0, forest1f, forest2f, fm)


def _tc_layer1(feat1, x2s, feat0, w1a, w1b16):
    N1, F = feat1.shape
    B = feat0.shape[0]
    R = 2048
    G = R // 16
    grid = N1 // R

    def body(f1_ref, x2_ref, f0_ref, wa_ref, wb_ref, h0_ref, h1s_ref):
        f1 = f1_ref[...]
        wa = wa_ref[...]
        wb = wb_ref[...]
        h1 = jnp.dot(f1, wa, preferred_element_type=jnp.float32)
        h1 = h1 + jnp.dot(x2_ref[...], wb, preferred_element_type=jnp.float32)
        h1 = jnp.maximum(h1, 0.0)
        h1s_ref[...] = h1.reshape(G, 16, F).sum(axis=1)
        xs = f1.reshape(G, 16, F).sum(axis=1)
        h0 = jnp.dot(f0_ref[...], wa, preferred_element_type=jnp.float32)
        h0 = h0 + jnp.dot(xs, wb, preferred_element_type=jnp.float32)
        h0_ref[...] = jnp.maximum(h0, 0.0)

    return pl.pallas_call(
        body,
        grid=(grid,),
        in_specs=[
            pl.BlockSpec((R, F), lambda i: (i, 0)),
            pl.BlockSpec((R, F), lambda i: (i, 0)),
            pl.BlockSpec((G, F), lambda i: (i, 0)),
            pl.BlockSpec((F, F), lambda i: (0, 0)),
            pl.BlockSpec((F, F), lambda i: (0, 0)),
        ],
        out_specs=[
            pl.BlockSpec((G, F), lambda i: (i, 0)),
            pl.BlockSpec((G, F), lambda i: (i, 0)),
        ],
        out_shape=[
            jax.ShapeDtypeStruct((B, F), jnp.float32),
            jax.ShapeDtypeStruct((B, F), jnp.float32),
        ],
    )(feat1, x2s, feat0, w1a, w1b16)


def _tc_layer2(h0, h1s, w2a, w2b16):
    B, H = h0.shape

    def body(h0_ref, h1_ref, wa_ref, wb_ref, out_ref):
        o = jnp.dot(h0_ref[...], wa_ref[...], preferred_element_type=jnp.float32)
        o = o + jnp.dot(h1_ref[...], wb_ref[...], preferred_element_type=jnp.float32)
        out_ref[...] = jnp.maximum(o, 0.0)

    return pl.pallas_call(
        body,
        out_shape=jax.ShapeDtypeStruct((B, H), jnp.float32),
    )(h0, h1s, w2a, w2b16)


def kernel(forest0, forest1, forest2, feature_matrix, W1, W2):
    F = feature_matrix.shape[1]
    f0 = forest0.astype(jnp.int32)
    f1 = forest1.reshape(-1).astype(jnp.int32)
    f2 = forest2.reshape(-1).astype(jnp.int32)

    feat0, feat1, x2s = _sc_gather_all(f0, f1, f2, feature_matrix)

    W1t = W1.T
    w1a = W1t[:F]
    w1b16 = W1t[F:] * (1.0 / 16.0)
    W2t = W2.T
    w2a = W2t[:F]
    w2b16 = W2t[F:] * (1.0 / 16.0)

    h0, h1s = _tc_layer1(feat1, x2s, feat0, w1a, w1b16)
    return _tc_layer2(h0, h1s, w2a, w2b16)
